# Initial kernel scaffold; baseline (speedup 1.0000x reference)
import functools

import jax
import jax.numpy as jnp
from jax import lax
from jax.experimental import pallas as pl
from jax.experimental.pallas import tpu as pltpu
from jax.experimental.pallas import tpu_sc as plsc

N = 10000
D = 128
CHUNK = 128
NC = 2
NS = 16
NW = NC * NS
PAD_ROWS = 16
ACC_ROWS = N + PAD_ROWS
ROWS_PER_TILE = ACC_ROWS // NS
LAST_TILE_ROWS = N - (NS - 1) * ROWS_PER_TILE


def _seg_sum_body(cpt, x_hbm, src_hbm, dst_hbm, out_hbm, acc, idx_s, idx_d, rows, sem):
    core = lax.axis_index("c")
    sub = lax.axis_index("s")
    wid = core * NS + sub

    def zrow(r, c0):
        def zcol(c, c1):
            rows[r, pl.ds(c * 16, 16)] = jnp.zeros((16,), jnp.float32)
            return c1
        return lax.fori_loop(0, D // 16, zcol, c0)
    lax.fori_loop(0, CHUNK, zrow, 0)

    base = sub * ROWS_PER_TILE
    for k in range(ROWS_PER_TILE // CHUNK):
        pltpu.sync_copy(rows, acc.at[pl.ds(base + k * CHUNK, CHUNK)])
    tail = ROWS_PER_TILE % CHUNK
    if tail:
        pltpu.sync_copy(rows.at[pl.ds(0, tail)],
                        acc.at[pl.ds(base + (ROWS_PER_TILE // CHUNK) * CHUNK, tail)])
    plsc.subcore_barrier()

    def body(i, carry):
        g = wid * cpt + i
        pltpu.sync_copy(src_hbm.at[g], idx_s)
        pltpu.sync_copy(dst_hbm.at[g], idx_d)
        pltpu.async_copy(x_hbm.at[idx_s], rows, sem).wait()
        pltpu.sync_copy(rows, acc.at[idx_d], add=True)
        return carry
    lax.fori_loop(0, cpt, body, 0)
    plsc.subcore_barrier()

    @pl.when(sub < NS - 1)
    def _():
        pltpu.sync_copy(acc.at[pl.ds(base, ROWS_PER_TILE)],
                        out_hbm.at[core, pl.ds(base, ROWS_PER_TILE)])

    @pl.when(sub == NS - 1)
    def _():
        pltpu.sync_copy(acc.at[pl.ds((NS - 1) * ROWS_PER_TILE, LAST_TILE_ROWS)],
                        out_hbm.at[core, pl.ds((NS - 1) * ROWS_PER_TILE, LAST_TILE_ROWS)])


def _sc_segment_sum(x, src_chunks, dst_chunks):
    cpt = src_chunks.shape[0] // NW
    mesh = plsc.VectorSubcoreMesh(core_axis_name="c", subcore_axis_name="s")
    f = pl.kernel(
        functools.partial(_seg_sum_body, cpt),
        out_type=jax.ShapeDtypeStruct((NC, N, D), jnp.float32),
        mesh=mesh,
        scratch_types=[
            pltpu.VMEM_SHARED((ACC_ROWS, D), jnp.float32),
            pltpu.VMEM((CHUNK,), jnp.int32),
            pltpu.VMEM((CHUNK,), jnp.int32),
            pltpu.VMEM((CHUNK, D), jnp.float32),
            pltpu.SemaphoreType.DMA,
        ],
    )
    return f(x, src_chunks, dst_chunks)


def _mlp_body(relu_out, x_ref, a0_ref, a1_ref, wa_ref, ba_ref, wb_ref, bb_ref, o_ref):
    h = x_ref[...] + a0_ref[...] + a1_ref[...]
    h = jnp.maximum(jnp.dot(h, wa_ref[...], preferred_element_type=jnp.float32)
                    + ba_ref[...], 0.0)
    o = jnp.dot(h, wb_ref[...], preferred_element_type=jnp.float32) + bb_ref[...]
    o_ref[...] = jnp.maximum(o, 0.0) if relu_out else o


def _tc_mlp(x, agg, wa, ba, wb, bb, relu_out):
    bn = 1000
    grid = (N // bn,)
    row_spec = pl.BlockSpec((bn, D), lambda i: (i, 0))
    full = pl.BlockSpec((D, D), lambda i: (0, 0))
    vec = pl.BlockSpec((1, D), lambda i: (0, 0))
    return pl.pallas_call(
        functools.partial(_mlp_body, relu_out),
        grid=grid,
        in_specs=[row_spec, row_spec, row_spec, full, vec, full, vec],
        out_specs=row_spec,
        out_shape=jax.ShapeDtypeStruct((N, D), jnp.float32),
    )(x, agg[0], agg[1], wa, ba.reshape(1, D), wb, bb.reshape(1, D))


def kernel(x_indices, ei, emb, W1a, b1a, W1b, b1b, W2a, b2a, W2b, b2b):
    x = jnp.take(emb, x_indices, axis=0)

    e = ei.shape[1]
    ep = -(-e // (NW * CHUNK)) * (NW * CHUNK)
    pad = ep - e
    src = jnp.concatenate([ei[0], jnp.zeros((pad,), jnp.int32)])
    dst = jnp.concatenate([ei[1], N + (jnp.arange(pad, dtype=jnp.int32) % PAD_ROWS)])
    src_chunks = src.reshape(-1, CHUNK)
    dst_chunks = dst.reshape(-1, CHUNK)

    agg1 = _sc_segment_sum(x, src_chunks, dst_chunks)
    x1 = _tc_mlp(x, agg1, W1a, b1a, W1b, b1b, relu_out=True)
    agg2 = _sc_segment_sum(x1, src_chunks, dst_chunks)
    return _tc_mlp(x1, agg2, W2a, b2a, W2b, b2b, relu_out=False)


# speedup vs baseline: 4.1211x; 4.1211x over previous
import functools

import jax
import jax.numpy as jnp
from jax import lax
from jax.experimental import pallas as pl
from jax.experimental.pallas import tpu as pltpu
from jax.experimental.pallas import tpu_sc as plsc

N = 10000
D = 128
CHUNK = 128
NC = 2
NS = 16
NW = NC * NS
PAD_ROWS = 240
ACC_ROWS = N + PAD_ROWS
ZERO_PER_TILE = ACC_ROWS // NS
OUT_PER_TILE = 624
LAST_TILE_ROWS = N - (NS - 1) * OUT_PER_TILE


def _seg_sum_body(cpt, x_hbm, src_hbm, dst_hbm, out_hbm, acc, idx_s, idx_d, rows, sem):
    core = lax.axis_index("c")
    sub = lax.axis_index("s")
    wid = core * NS + sub

    def zrow(r, c0):
        def zcol(c, c1):
            rows[r, pl.ds(c * 16, 16)] = jnp.zeros((16,), jnp.float32)
            return c1
        return lax.fori_loop(0, D // 16, zcol, c0)
    lax.fori_loop(0, CHUNK, zrow, 0)

    zbase = sub * ZERO_PER_TILE
    for k in range(ZERO_PER_TILE // CHUNK):
        pltpu.sync_copy(rows, acc.at[pl.ds(zbase + k * CHUNK, CHUNK)])
    plsc.subcore_barrier()

    def body(i, carry):
        g = wid * cpt + i
        pltpu.sync_copy(src_hbm.at[g], idx_s)
        pltpu.sync_copy(dst_hbm.at[g], idx_d)
        pltpu.async_copy(x_hbm.at[idx_s], rows, sem).wait()
        pltpu.sync_copy(rows, acc.at[idx_d], add=True)
        return carry
    lax.fori_loop(0, cpt, body, 0)
    plsc.subcore_barrier()

    obase = sub * OUT_PER_TILE
    @pl.when(sub < NS - 1)
    def _():
        pltpu.sync_copy(acc.at[pl.ds(obase, OUT_PER_TILE)],
                        out_hbm.at[core, pl.ds(obase, OUT_PER_TILE)])

    @pl.when(sub == NS - 1)
    def _():
        pltpu.sync_copy(acc.at[pl.ds((NS - 1) * OUT_PER_TILE, LAST_TILE_ROWS)],
                        out_hbm.at[core, pl.ds((NS - 1) * OUT_PER_TILE, LAST_TILE_ROWS)])


def _sc_segment_sum(x, src_chunks, dst_chunks):
    cpt = src_chunks.shape[0] // NW
    mesh = plsc.VectorSubcoreMesh(core_axis_name="c", subcore_axis_name="s")
    f = pl.kernel(
        functools.partial(_seg_sum_body, cpt),
        out_type=jax.ShapeDtypeStruct((NC, N, D), jnp.float32),
        mesh=mesh,
        scratch_types=[
            pltpu.VMEM_SHARED((ACC_ROWS, D), jnp.float32),
            pltpu.VMEM((CHUNK,), jnp.int32),
            pltpu.VMEM((CHUNK,), jnp.int32),
            pltpu.VMEM((CHUNK, D), jnp.float32),
            pltpu.SemaphoreType.DMA,
        ],
    )
    return f(x, src_chunks, dst_chunks)


def _mlp_body(relu_out, x_ref, a0_ref, a1_ref, wa_ref, ba_ref, wb_ref, bb_ref, o_ref):
    h = x_ref[...] + a0_ref[...] + a1_ref[...]
    h = jnp.maximum(jnp.dot(h, wa_ref[...], preferred_element_type=jnp.float32)
                    + ba_ref[...], 0.0)
    o = jnp.dot(h, wb_ref[...], preferred_element_type=jnp.float32) + bb_ref[...]
    o_ref[...] = jnp.maximum(o, 0.0) if relu_out else o


def _tc_mlp(x, agg, wa, ba, wb, bb, relu_out):
    bn = 1000
    grid = (N // bn,)
    row_spec = pl.BlockSpec((bn, D), lambda i: (i, 0))
    full = pl.BlockSpec((D, D), lambda i: (0, 0))
    vec = pl.BlockSpec((1, D), lambda i: (0, 0))
    return pl.pallas_call(
        functools.partial(_mlp_body, relu_out),
        grid=grid,
        in_specs=[row_spec, row_spec, row_spec, full, vec, full, vec],
        out_specs=row_spec,
        out_shape=jax.ShapeDtypeStruct((N, D), jnp.float32),
    )(x, agg[0], agg[1], wa, ba.reshape(1, D), wb, bb.reshape(1, D))


def kernel(x_indices, ei, emb, W1a, b1a, W1b, b1b, W2a, b2a, W2b, b2b):
    x = jnp.take(emb, x_indices, axis=0)

    e = ei.shape[1]
    ep = -(-e // (NW * CHUNK)) * (NW * CHUNK)
    pad = ep - e
    src = jnp.concatenate([ei[0], jnp.zeros((pad,), jnp.int32)])
    dst = jnp.concatenate([ei[1], N + (jnp.arange(pad, dtype=jnp.int32) % PAD_ROWS)])
    src_chunks = src.reshape(-1, CHUNK)
    dst_chunks = dst.reshape(-1, CHUNK)

    agg1 = _sc_segment_sum(x, src_chunks, dst_chunks)
    x1 = _tc_mlp(x, agg1, W1a, b1a, W1b, b1b, relu_out=True)
    agg2 = _sc_segment_sum(x1, src_chunks, dst_chunks)
    return _tc_mlp(x1, agg2, W2a, b2a, W2b, b2b, relu_out=False)
